# baseline (speedup 1.0000x reference)
import jax
import jax.numpy as jnp
from jax import lax
from jax.experimental import pallas as pl

_BATCH = 16384
_BM = 4096


def _mlp_body(f_ref, idx_ref, t_ref, w1_ref, w2_ref, b2_ref, w3d_ref, b3d_ref,
              wd2_ref, bd2_ref, emb_ref, rec_ref):
    idxb = jnp.broadcast_to(idx_ref[...], (8, _BM))
    iota = lax.broadcasted_iota(jnp.int32, (8, _BM), 0)
    onehot_t = (idxb == iota).astype(jnp.bfloat16)
    g = lax.dot_general(onehot_t, t_ref[...], (((0,), (0,)), ((), ())),
                        preferred_element_type=jnp.float32)
    f16 = f_ref[...].astype(jnp.bfloat16)
    h1 = lax.dot_general(f16, w1_ref[...], (((1,), (0,)), ((), ())),
                         preferred_element_type=jnp.float32)
    h1 = jnp.maximum(h1 + g, 0.0).astype(jnp.bfloat16)
    h2 = lax.dot_general(h1, w2_ref[...], (((1,), (0,)), ((), ())),
                         preferred_element_type=jnp.float32)
    h2 = jnp.maximum(h2 + b2_ref[...], 0.0).astype(jnp.bfloat16)
    ed = lax.dot_general(h2, w3d_ref[...], (((1,), (0,)), ((), ())),
                         preferred_element_type=jnp.float32) + b3d_ref[...]
    emb_ref[...] = ed[:, :16]
    d = jnp.maximum(ed[:, 16:], 0.0).astype(jnp.bfloat16)
    rec_ref[...] = lax.dot_general(d, wd2_ref[...], (((1,), (0,)), ((), ())),
                                   preferred_element_type=jnp.float32) + bd2_ref[...]


def kernel(features, position_idx, pos_emb, W1, b1, W2, b2, W3, b3,
           Wd1, bd1, Wd2, bd2):
    T = pos_emb @ W1[128:] + b1
    T8 = jnp.zeros((8, 32), jnp.float32).at[:T.shape[0]].set(T).astype(jnp.bfloat16)
    W1a = W1[:128].astype(jnp.bfloat16)
    W3d = jnp.concatenate([W3, W3 @ Wd1], axis=1).astype(jnp.bfloat16)
    b3d = jnp.concatenate([b3, b3 @ Wd1 + bd1])[None, :]
    idx2d = position_idx.reshape(1, _BATCH)

    nb = _BATCH // _BM
    full = lambda shape: pl.BlockSpec(shape, lambda i: (0, 0))
    emb, rec = pl.pallas_call(
        _mlp_body,
        grid=(nb,),
        in_specs=[
            pl.BlockSpec((_BM, 128), lambda i: (i, 0)),
            pl.BlockSpec((1, _BM), lambda i: (0, i)),
            full((8, 32)),
            full((128, 32)),
            full((32, 16)),
            full((1, 16)),
            full((16, 32)),
            full((1, 32)),
            full((16, 128)),
            full((1, 128)),
        ],
        out_specs=[
            pl.BlockSpec((_BM, 16), lambda i: (i, 0)),
            pl.BlockSpec((_BM, 128), lambda i: (i, 0)),
        ],
        out_shape=[
            jax.ShapeDtypeStruct((_BATCH, 16), jnp.float32),
            jax.ShapeDtypeStruct((_BATCH, 128), jnp.float32),
        ],
    )(features, idx2d, T8, W1a, W2.astype(jnp.bfloat16),
      b2[None, :], W3d, b3d, Wd2.astype(jnp.bfloat16), bd2[None, :])
    return (emb, rec)


# speedup vs baseline: 1.3323x; 1.3323x over previous
import jax
import jax.numpy as jnp
from jax import lax
from jax.experimental import pallas as pl

_BATCH = 16384
_BM = 8192


def _dot(a, b):
    return lax.dot_general(a, b, (((1,), (0,)), ((), ())),
                           preferred_element_type=jnp.float32)


def _mlp_body(f_ref, idx_ref, pe_ref, w1_ref, b1_ref, w2_ref, b2_ref,
              w3_ref, b3_ref, wd1_ref, bd1_ref, wd2_ref, bd2_ref,
              emb_ref, rec_ref):
    w1b = w1_ref[128:136, :]
    t = _dot(pe_ref[...].astype(jnp.bfloat16),
             w1b.astype(jnp.bfloat16)) + b1_ref[...]
    t = jnp.concatenate([t, jnp.zeros((2, 32), jnp.float32)], axis=0)
    w3d = jnp.concatenate(
        [w3_ref[...],
         _dot(w3_ref[...].astype(jnp.bfloat16),
              wd1_ref[...].astype(jnp.bfloat16))], axis=1)
    b3d = jnp.concatenate(
        [b3_ref[...],
         _dot(b3_ref[...].astype(jnp.bfloat16),
              wd1_ref[...].astype(jnp.bfloat16)) + bd1_ref[...]], axis=1)

    idxb = jnp.broadcast_to(idx_ref[...], (8, _BM))
    iota = lax.broadcasted_iota(jnp.int32, (8, _BM), 0)
    onehot_t = (idxb == iota).astype(jnp.bfloat16)
    g = lax.dot_general(onehot_t, t.astype(jnp.bfloat16),
                        (((0,), (0,)), ((), ())),
                        preferred_element_type=jnp.float32)

    h1 = _dot(f_ref[...].astype(jnp.bfloat16), w1_ref[:128, :].astype(jnp.bfloat16))
    h1 = jnp.maximum(h1 + g, 0.0).astype(jnp.bfloat16)
    h2 = _dot(h1, w2_ref[...].astype(jnp.bfloat16))
    h2 = jnp.maximum(h2 + b2_ref[...], 0.0).astype(jnp.bfloat16)
    ed = _dot(h2, w3d.astype(jnp.bfloat16)) + b3d
    emb_ref[...] = ed[:, :16]
    d = jnp.maximum(ed[:, 16:], 0.0).astype(jnp.bfloat16)
    rec_ref[...] = _dot(d, wd2_ref[...].astype(jnp.bfloat16)) + bd2_ref[...]


def kernel(features, position_idx, pos_emb, W1, b1, W2, b2, W3, b3,
           Wd1, bd1, Wd2, bd2):
    idx2d = position_idx.reshape(1, _BATCH)

    nb = _BATCH // _BM
    full = lambda shape: pl.BlockSpec(shape, lambda i: (0, 0))
    emb, rec = pl.pallas_call(
        _mlp_body,
        grid=(nb,),
        in_specs=[
            pl.BlockSpec((_BM, 128), lambda i: (i, 0)),
            pl.BlockSpec((1, _BM), lambda i: (0, i)),
            full((6, 8)),
            full((136, 32)),
            full((1, 32)),
            full((32, 16)),
            full((1, 16)),
            full((16, 16)),
            full((1, 16)),
            full((16, 16)),
            full((1, 16)),
            full((16, 128)),
            full((1, 128)),
        ],
        out_specs=[
            pl.BlockSpec((_BM, 16), lambda i: (i, 0)),
            pl.BlockSpec((_BM, 128), lambda i: (i, 0)),
        ],
        out_shape=[
            jax.ShapeDtypeStruct((_BATCH, 16), jnp.float32),
            jax.ShapeDtypeStruct((_BATCH, 128), jnp.float32),
        ],
    )(features, idx2d, pos_emb, W1, b1[None, :], W2, b2[None, :], W3, b3[None, :],
      Wd1, bd1[None, :], Wd2, bd2[None, :])
    return (emb, rec)
